# baseline (speedup 1.0000x reference)
import functools

import jax
import jax.numpy as jnp
from jax import lax
from jax.experimental import pallas as pl
from jax.experimental.pallas import tpu as pltpu
from jax.experimental.pallas import tpu_sc as plsc

N_NODES = 10000
F = 128
HID = 256
K = 10
ALPHA = 0.1

N_PAD = 10112
NW = 32
EDGES_PER_TILE = 10000
EB = 128
NB = 81
EDGES_PAD_PER_TILE = NB * EB
ROWS_PER_TILE = N_PAD // 16


def _sc_scatter_body(g_hbm, pair_hbm, zeros_hbm, out_hbm,
                     accum, p0, p1, p2, r0, r1, r2,
                     sg0, sg1, sg2, ss0, ss1, ss2, sp0, sp1, sp2):
    pairs = [p0, p1, p2]
    rows = [r0, r1, r2]
    sem_g = [sg0, sg1, sg2]
    sem_s = [ss0, ss1, ss2]
    sem_p = [sp0, sp1, sp2]
    c = lax.axis_index("c")
    s = lax.axis_index("s")
    wid = c * 16 + s
    base = wid * NB

    zcp = pltpu.async_copy(
        zeros_hbm.at[pl.ds(s * ROWS_PER_TILE, ROWS_PER_TILE)],
        accum.at[pl.ds(s * ROWS_PER_TILE, ROWS_PER_TILE)], ss0)
    pltpu.sync_copy(pair_hbm.at[base], p0)
    pltpu.sync_copy(pair_hbm.at[base + 1], p1)
    astart = pltpu.async_copy(pair_hbm.at[base + 2], p2, sp2)
    zcp.wait()
    plsc.subcore_barrier()

    def gather(k):
        pltpu.async_copy(g_hbm.at[pairs[k].at[0]], rows[k], sem_g[k])

    def gwait(k):
        pltpu.make_async_copy(g_hbm.at[pairs[k].at[0]], rows[k],
                              sem_g[k]).wait()

    def scatter(k):
        pltpu.async_copy(rows[k], accum.at[pairs[k].at[1]], sem_s[k],
                         add=True)

    def swait(k):
        pltpu.make_async_copy(rows[k], accum.at[pairs[k].at[1]],
                              sem_s[k]).wait()

    def astage(k, batch):
        b = jnp.minimum(batch, NB - 1)
        pltpu.async_copy(pair_hbm.at[base + b], pairs[k], sem_p[k])

    def pwait(k):
        pltpu.make_async_copy(pair_hbm.at[base], pairs[k], sem_p[k]).wait()

    gather(0)
    gather(1)

    def body(i, carry):
        b = 3 * i
        gwait(0)
        pwait(2)
        gather(2)
        scatter(0)
        gwait(1)
        scatter(1)
        swait(0)
        astage(0, b + 3)
        gwait(2)
        pwait(0)
        gather(0)
        scatter(2)
        swait(1)
        astage(1, b + 4)
        pwait(1)
        gather(1)
        swait(2)
        astage(2, b + 5)
        return carry

    lax.fori_loop(0, NB // 3, body, 0)
    gwait(0)
    gwait(1)
    pwait(2)

    plsc.subcore_barrier()
    pltpu.sync_copy(accum.at[pl.ds(s * ROWS_PER_TILE, ROWS_PER_TILE)],
                    out_hbm.at[c, pl.ds(s * ROWS_PER_TILE, ROWS_PER_TILE)])


_sc_scatter = pl.kernel(
    _sc_scatter_body,
    mesh=plsc.VectorSubcoreMesh(core_axis_name="c", subcore_axis_name="s"),
    out_type=jax.ShapeDtypeStruct((2, N_PAD, F), jnp.float32),
    scratch_types=(
        [pltpu.VMEM_SHARED((N_PAD, F), jnp.float32)]
        + [pltpu.VMEM((2, EB), jnp.int32)] * 3
        + [pltpu.VMEM((EB, F), jnp.float32)] * 3
        + [pltpu.SemaphoreType.DMA] * 9
    ),
)


DF = 16


def _sc_deg_body(pair_hbm, zeros_hbm, ones_hbm, out_hbm,
                 accum, p0, p1, ones_rows, ss0, ss1, sp0, sp1):
    pairs = [p0, p1]
    sem_s = [ss0, ss1]
    sem_p = [sp0, sp1]
    c = lax.axis_index("c")
    s = lax.axis_index("s")
    wid = c * 16 + s
    base = wid * NB

    zcp = pltpu.async_copy(
        zeros_hbm.at[pl.ds(s * ROWS_PER_TILE, ROWS_PER_TILE)],
        accum.at[pl.ds(s * ROWS_PER_TILE, ROWS_PER_TILE)], ss0)
    pltpu.sync_copy(ones_hbm, ones_rows)
    pltpu.sync_copy(pair_hbm.at[base], p0)
    pltpu.async_copy(pair_hbm.at[base + 1], p1, sp1)
    zcp.wait()
    plsc.subcore_barrier()

    def scatter(k):
        pltpu.async_copy(ones_rows, accum.at[pairs[k].at[1]], sem_s[k],
                         add=True)

    def swait(k):
        pltpu.make_async_copy(ones_rows, accum.at[pairs[k].at[1]],
                              sem_s[k]).wait()

    def astage(k, batch):
        b = jnp.minimum(batch, NB - 1)
        pltpu.async_copy(pair_hbm.at[base + b], pairs[k], sem_p[k])

    def pwait(k):
        pltpu.make_async_copy(pair_hbm.at[base], pairs[k], sem_p[k]).wait()

    scatter(0)

    def body(i, carry):
        b = 2 * i
        swait(0)
        astage(0, b + 2)
        pwait(1)
        scatter(1)
        swait(1)
        astage(1, b + 3)
        pwait(0)
        scatter(0)
        return carry

    lax.fori_loop(0, NB // 2, body, 0)
    swait(0)
    pwait(1)

    plsc.subcore_barrier()
    pltpu.sync_copy(accum.at[pl.ds(s * ROWS_PER_TILE, ROWS_PER_TILE)],
                    out_hbm.at[c, pl.ds(s * ROWS_PER_TILE, ROWS_PER_TILE)])


_sc_deg = pl.kernel(
    _sc_deg_body,
    mesh=plsc.VectorSubcoreMesh(core_axis_name="c", subcore_axis_name="s"),
    out_type=jax.ShapeDtypeStruct((2, N_PAD, DF), jnp.float32),
    scratch_types=(
        [pltpu.VMEM_SHARED((N_PAD, DF), jnp.float32)]
        + [pltpu.VMEM((2, EB), jnp.int32)] * 2
        + [pltpu.VMEM((EB, DF), jnp.float32)]
        + [pltpu.SemaphoreType.DMA] * 4
    ),
)


def _mlp_body(x_ref, w1_ref, b1_ref, w2_ref, b2_ref, da_ref, db_ref,
              h0_ref, g0_ref, q_ref, dis_ref):
    deg = da_ref[:, 0:1] + db_ref[:, 0:1] + 1.0
    dis = lax.rsqrt(deg)
    h = jnp.maximum(
        jnp.dot(x_ref[...], w1_ref[...], preferred_element_type=jnp.float32)
        + b1_ref[...], 0.0)
    h = jnp.dot(h, w2_ref[...], preferred_element_type=jnp.float32) + b2_ref[...]
    h0_ref[...] = h
    g0_ref[...] = dis * h
    q_ref[...] = 1.0 / deg
    dis_ref[...] = dis


def _update_body(sa_ref, sb_ref, g_ref, base_ref, vec_ref, out_ref):
    out_ref[...] = ((1.0 - ALPHA) * vec_ref[...]
                    * (sa_ref[...] + sb_ref[...] + g_ref[...])
                    + ALPHA * base_ref[...])


_BLK = 632
_GRID = N_PAD // _BLK


def _row_spec(width):
    return pl.BlockSpec((_BLK, width), lambda i: (i, 0))


_mlp_call = pl.pallas_call(
    _mlp_body,
    grid=(_GRID,),
    in_specs=[
        _row_spec(F),
        pl.BlockSpec((F, HID), lambda i: (0, 0)),
        pl.BlockSpec((1, HID), lambda i: (0, 0)),
        pl.BlockSpec((HID, F), lambda i: (0, 0)),
        pl.BlockSpec((1, F), lambda i: (0, 0)),
        _row_spec(DF),
        _row_spec(DF),
    ],
    out_specs=[_row_spec(F), _row_spec(F), _row_spec(1), _row_spec(1)],
    out_shape=[
        jax.ShapeDtypeStruct((N_PAD, F), jnp.float32),
        jax.ShapeDtypeStruct((N_PAD, F), jnp.float32),
        jax.ShapeDtypeStruct((N_PAD, 1), jnp.float32),
        jax.ShapeDtypeStruct((N_PAD, 1), jnp.float32),
    ],
)

_update_call = pl.pallas_call(
    _update_body,
    grid=(_GRID,),
    in_specs=[_row_spec(F), _row_spec(F), _row_spec(F), _row_spec(F),
              _row_spec(1)],
    out_specs=_row_spec(F),
    out_shape=jax.ShapeDtypeStruct((N_PAD, F), jnp.float32),
)


def kernel(x, edge_index, W1, b1, W2, b2):
    src = edge_index[0].astype(jnp.int32).reshape(NW, EDGES_PER_TILE)
    dst = edge_index[1].astype(jnp.int32).reshape(NW, EDGES_PER_TILE)

    n_extra = EDGES_PAD_PER_TILE - EDGES_PER_TILE
    w_ids = jnp.arange(NW, dtype=jnp.int32)[:, None]
    j_ids = jnp.arange(n_extra, dtype=jnp.int32)[None, :]
    pad_src = (w_ids * 37 + j_ids * 89) % N_NODES
    pad_dst = N_NODES + (w_ids * 13 + j_ids * 7) % (N_PAD - N_NODES)
    src_t = jnp.concatenate([src, pad_src], axis=1).reshape(NW * NB, EB)
    dst_t = jnp.concatenate([dst, pad_dst], axis=1).reshape(NW * NB, EB)
    pair = jnp.stack([src_t, dst_t], axis=1)

    zeros = jnp.zeros((N_PAD, F), jnp.float32)
    zeros16 = jnp.zeros((N_PAD, DF), jnp.float32)
    ones16 = jnp.ones((EB, DF), jnp.float32)

    deg_parts_wide = _sc_scatter(jnp.ones((N_PAD, F), jnp.float32), pair,
                                 zeros)
    deg_parts = deg_parts_wide[:, :, :DF]

    x_pad = jnp.zeros((N_PAD, F), x.dtype).at[:N_NODES].set(x)
    h0, g0, q, dis = _mlp_call(x_pad, W1, b1.reshape(1, HID), W2,
                               b2.reshape(1, F), deg_parts[0], deg_parts[1])

    g = g0
    for k in range(K):
        s_parts = _sc_scatter(g, pair, zeros)
        if k < K - 1:
            g = _update_call(s_parts[0], s_parts[1], g, g0, q)
        else:
            h = _update_call(s_parts[0], s_parts[1], g, h0, dis)
    return h[:N_NODES]


# speedup vs baseline: 1.0401x; 1.0401x over previous
import functools

import jax
import jax.numpy as jnp
from jax import lax
from jax.experimental import pallas as pl
from jax.experimental.pallas import tpu as pltpu
from jax.experimental.pallas import tpu_sc as plsc

N_NODES = 10000
F = 128
HID = 256
K = 10
ALPHA = 0.1

N_PAD = 10112
NW = 32
EDGES_PER_TILE = 10000
EB = 128
NB = 81
EDGES_PAD_PER_TILE = NB * EB
ROWS_PER_TILE = N_PAD // 16


def _sc_scatter_body(g_hbm, pair_hbm, zeros_hbm, out_hbm,
                     accum, p0, p1, p2, r0, r1, r2,
                     sg0, sg1, sg2, ss0, ss1, ss2, sp0, sp1, sp2):
    pairs = [p0, p1, p2]
    rows = [r0, r1, r2]
    sem_g = [sg0, sg1, sg2]
    sem_s = [ss0, ss1, ss2]
    sem_p = [sp0, sp1, sp2]
    c = lax.axis_index("c")
    s = lax.axis_index("s")
    wid = c * 16 + s
    base = wid * NB

    zcp = pltpu.async_copy(
        zeros_hbm.at[pl.ds(s * ROWS_PER_TILE, ROWS_PER_TILE)],
        accum.at[pl.ds(s * ROWS_PER_TILE, ROWS_PER_TILE)], ss0)
    pltpu.sync_copy(pair_hbm.at[base], p0)
    pltpu.sync_copy(pair_hbm.at[base + 1], p1)
    astart = pltpu.async_copy(pair_hbm.at[base + 2], p2, sp2)
    zcp.wait()
    plsc.subcore_barrier()

    def gather(k):
        pltpu.async_copy(g_hbm.at[pairs[k].at[0]], rows[k], sem_g[k])

    def gwait(k):
        pltpu.make_async_copy(g_hbm.at[pairs[k].at[0]], rows[k],
                              sem_g[k]).wait()

    def scatter(k):
        pltpu.async_copy(rows[k], accum.at[pairs[k].at[1]], sem_s[k],
                         add=True)

    def swait(k):
        pltpu.make_async_copy(rows[k], accum.at[pairs[k].at[1]],
                              sem_s[k]).wait()

    def astage(k, batch):
        b = jnp.minimum(batch, NB - 1)
        pltpu.async_copy(pair_hbm.at[base + b], pairs[k], sem_p[k])

    def pwait(k):
        pltpu.make_async_copy(pair_hbm.at[base], pairs[k], sem_p[k]).wait()

    gather(0)
    gather(1)

    def body(i, carry):
        b = 3 * i
        gwait(0)
        pwait(2)
        gather(2)
        scatter(0)
        gwait(1)
        swait(0)
        astage(0, b + 3)
        scatter(1)
        gwait(2)
        pwait(0)
        gather(0)
        swait(1)
        astage(1, b + 4)
        scatter(2)
        pwait(1)
        gather(1)
        swait(2)
        astage(2, b + 5)
        return carry

    lax.fori_loop(0, NB // 3, body, 0)
    gwait(0)
    gwait(1)
    pwait(2)

    plsc.subcore_barrier()
    pltpu.sync_copy(accum.at[pl.ds(s * ROWS_PER_TILE, ROWS_PER_TILE)],
                    out_hbm.at[c, pl.ds(s * ROWS_PER_TILE, ROWS_PER_TILE)])


_sc_scatter = pl.kernel(
    _sc_scatter_body,
    mesh=plsc.VectorSubcoreMesh(core_axis_name="c", subcore_axis_name="s"),
    out_type=jax.ShapeDtypeStruct((2, N_PAD, F), jnp.float32),
    scratch_types=(
        [pltpu.VMEM_SHARED((N_PAD, F), jnp.float32)]
        + [pltpu.VMEM((2, EB), jnp.int32)] * 3
        + [pltpu.VMEM((EB, F), jnp.float32)] * 3
        + [pltpu.SemaphoreType.DMA] * 9
    ),
)


DF = 128


def _sc_deg_body(pair_hbm, zeros_hbm, ones_hbm, out_hbm,
                 accum, p0, p1, ones_rows, ss0, ss1, sp0, sp1):
    pairs = [p0, p1]
    sem_s = [ss0, ss1]
    sem_p = [sp0, sp1]
    c = lax.axis_index("c")
    s = lax.axis_index("s")
    wid = c * 16 + s
    base = wid * NB

    zcp = pltpu.async_copy(
        zeros_hbm.at[pl.ds(s * ROWS_PER_TILE, ROWS_PER_TILE)],
        accum.at[pl.ds(s * ROWS_PER_TILE, ROWS_PER_TILE)], ss0)
    pltpu.sync_copy(ones_hbm, ones_rows)
    pltpu.sync_copy(pair_hbm.at[base], p0)
    pltpu.async_copy(pair_hbm.at[base + 1], p1, sp1)
    zcp.wait()
    plsc.subcore_barrier()

    def scatter(k):
        pltpu.async_copy(ones_rows, accum.at[pairs[k].at[1]], sem_s[k],
                         add=True)

    def swait(k):
        pltpu.make_async_copy(ones_rows, accum.at[pairs[k].at[1]],
                              sem_s[k]).wait()

    def astage(k, batch):
        b = jnp.minimum(batch, NB - 1)
        pltpu.async_copy(pair_hbm.at[base + b], pairs[k], sem_p[k])

    def pwait(k):
        pltpu.make_async_copy(pair_hbm.at[base], pairs[k], sem_p[k]).wait()

    scatter(0)

    def body(i, carry):
        b = 2 * i
        swait(0)
        astage(0, b + 2)
        pwait(1)
        scatter(1)
        swait(1)
        astage(1, b + 3)
        pwait(0)
        scatter(0)
        return carry

    lax.fori_loop(0, NB // 2, body, 0)
    swait(0)
    pwait(1)

    plsc.subcore_barrier()
    pltpu.sync_copy(accum.at[pl.ds(s * ROWS_PER_TILE, ROWS_PER_TILE)],
                    out_hbm.at[c, pl.ds(s * ROWS_PER_TILE, ROWS_PER_TILE)])


_sc_deg = pl.kernel(
    _sc_deg_body,
    mesh=plsc.VectorSubcoreMesh(core_axis_name="c", subcore_axis_name="s"),
    out_type=jax.ShapeDtypeStruct((2, N_PAD, DF), jnp.float32),
    scratch_types=(
        [pltpu.VMEM_SHARED((N_PAD, DF), jnp.float32)]
        + [pltpu.VMEM((2, EB), jnp.int32)] * 2
        + [pltpu.VMEM((EB, DF), jnp.float32)]
        + [pltpu.SemaphoreType.DMA] * 4
    ),
)


def _mlp_body(x_ref, w1_ref, b1_ref, w2_ref, b2_ref, da_ref, db_ref,
              h0_ref, g0_ref, q_ref, dis_ref):
    deg = da_ref[:, 0:1] + db_ref[:, 0:1] + 1.0
    dis = lax.rsqrt(deg)
    h = jnp.maximum(
        jnp.dot(x_ref[...], w1_ref[...], preferred_element_type=jnp.float32)
        + b1_ref[...], 0.0)
    h = jnp.dot(h, w2_ref[...], preferred_element_type=jnp.float32) + b2_ref[...]
    h0_ref[...] = h
    g0_ref[...] = dis * h
    q_ref[...] = 1.0 / deg
    dis_ref[...] = dis


def _update_body(sa_ref, sb_ref, g_ref, base_ref, vec_ref, out_ref):
    out_ref[...] = ((1.0 - ALPHA) * vec_ref[...]
                    * (sa_ref[...] + sb_ref[...] + g_ref[...])
                    + ALPHA * base_ref[...])


_BLK = 632
_GRID = N_PAD // _BLK


def _row_spec(width):
    return pl.BlockSpec((_BLK, width), lambda i: (i, 0))


_mlp_call = pl.pallas_call(
    _mlp_body,
    grid=(_GRID,),
    in_specs=[
        _row_spec(F),
        pl.BlockSpec((F, HID), lambda i: (0, 0)),
        pl.BlockSpec((1, HID), lambda i: (0, 0)),
        pl.BlockSpec((HID, F), lambda i: (0, 0)),
        pl.BlockSpec((1, F), lambda i: (0, 0)),
        _row_spec(DF),
        _row_spec(DF),
    ],
    out_specs=[_row_spec(F), _row_spec(F), _row_spec(1), _row_spec(1)],
    out_shape=[
        jax.ShapeDtypeStruct((N_PAD, F), jnp.float32),
        jax.ShapeDtypeStruct((N_PAD, F), jnp.float32),
        jax.ShapeDtypeStruct((N_PAD, 1), jnp.float32),
        jax.ShapeDtypeStruct((N_PAD, 1), jnp.float32),
    ],
)

_update_call = pl.pallas_call(
    _update_body,
    grid=(_GRID,),
    in_specs=[_row_spec(F), _row_spec(F), _row_spec(F), _row_spec(F),
              _row_spec(1)],
    out_specs=_row_spec(F),
    out_shape=jax.ShapeDtypeStruct((N_PAD, F), jnp.float32),
)


def kernel(x, edge_index, W1, b1, W2, b2):
    src = edge_index[0].astype(jnp.int32).reshape(NW, EDGES_PER_TILE)
    dst = edge_index[1].astype(jnp.int32).reshape(NW, EDGES_PER_TILE)

    n_extra = EDGES_PAD_PER_TILE - EDGES_PER_TILE
    w_ids = jnp.arange(NW, dtype=jnp.int32)[:, None]
    j_ids = jnp.arange(n_extra, dtype=jnp.int32)[None, :]
    pad_src = (w_ids * 37 + j_ids * 89) % N_NODES
    pad_dst = N_NODES + (w_ids * 13 + j_ids * 7) % (N_PAD - N_NODES)
    src_t = jnp.concatenate([src, pad_src], axis=1).reshape(NW * NB, EB)
    dst_t = jnp.concatenate([dst, pad_dst], axis=1).reshape(NW * NB, EB)
    pair = jnp.stack([src_t, dst_t], axis=1)

    zeros = jnp.zeros((N_PAD, F), jnp.float32)
    zeros16 = jnp.zeros((N_PAD, DF), jnp.float32)
    ones16 = jnp.ones((EB, DF), jnp.float32)

    deg_parts = _sc_deg(pair, zeros16, ones16)

    x_pad = jnp.zeros((N_PAD, F), x.dtype).at[:N_NODES].set(x)
    h0, g0, q, dis = _mlp_call(x_pad, W1, b1.reshape(1, HID), W2,
                               b2.reshape(1, F), deg_parts[0], deg_parts[1])

    g = g0
    for k in range(K):
        s_parts = _sc_scatter(g, pair, zeros)
        if k < K - 1:
            g = _update_call(s_parts[0], s_parts[1], g, g0, q)
        else:
            h = _update_call(s_parts[0], s_parts[1], g, h0, dis)
    return h[:N_NODES]


# speedup vs baseline: 1.0419x; 1.0017x over previous
import jax
import jax.numpy as jnp
from jax import lax
from jax.experimental import pallas as pl
from jax.experimental.pallas import tpu as pltpu
from jax.experimental.pallas import tpu_sc as plsc

N_NODES = 10000
F = 128
HID = 256
K = 10
ALPHA = 0.1

N_PAD = 10112
NW = 32
EDGES_PER_TILE = 10000
EB = 128
NB = 81
EDGES_PAD_PER_TILE = NB * EB
ROWS_PER_TILE = N_PAD // 16


def _sc_scatter_body(g_hbm, pair_hbm, zeros_hbm, out_hbm,
                     accum, p0, p1, p2, r0, r1, r2,
                     sg0, sg1, sg2, ss0, ss1, ss2, sp0, sp1, sp2):
    pairs = [p0, p1, p2]
    rows = [r0, r1, r2]
    sem_g = [sg0, sg1, sg2]
    sem_s = [ss0, ss1, ss2]
    sem_p = [sp0, sp1, sp2]
    c = lax.axis_index("c")
    s = lax.axis_index("s")
    wid = c * 16 + s
    base = wid * NB

    zcp = pltpu.async_copy(
        zeros_hbm.at[pl.ds(s * ROWS_PER_TILE, ROWS_PER_TILE)],
        accum.at[pl.ds(s * ROWS_PER_TILE, ROWS_PER_TILE)], ss0)
    pltpu.sync_copy(pair_hbm.at[base], p0)
    pltpu.sync_copy(pair_hbm.at[base + 1], p1)
    pltpu.async_copy(pair_hbm.at[base + 2], p2, sp2)
    zcp.wait()
    plsc.subcore_barrier()

    def gather(k):
        pltpu.async_copy(g_hbm.at[pairs[k].at[0]], rows[k], sem_g[k])

    def gwait(k):
        pltpu.make_async_copy(g_hbm.at[pairs[k].at[0]], rows[k],
                              sem_g[k]).wait()

    def scatter(k):
        pltpu.async_copy(rows[k], accum.at[pairs[k].at[1]], sem_s[k],
                         add=True)

    def swait(k):
        pltpu.make_async_copy(rows[k], accum.at[pairs[k].at[1]],
                              sem_s[k]).wait()

    def astage(k, batch):
        b = jnp.minimum(batch, NB - 1)
        pltpu.async_copy(pair_hbm.at[base + b], pairs[k], sem_p[k])

    def pwait(k):
        pltpu.make_async_copy(pair_hbm.at[base], pairs[k], sem_p[k]).wait()

    gather(0)
    gather(1)

    def body(i, carry):
        b = 3 * i
        gwait(0)
        pwait(2)
        gather(2)
        scatter(0)
        gwait(1)
        swait(0)
        astage(0, b + 3)
        scatter(1)
        gwait(2)
        pwait(0)
        gather(0)
        swait(1)
        astage(1, b + 4)
        scatter(2)
        pwait(1)
        gather(1)
        swait(2)
        astage(2, b + 5)
        return carry

    lax.fori_loop(0, NB // 3, body, 0)
    gwait(0)
    gwait(1)
    pwait(2)

    plsc.subcore_barrier()
    pltpu.sync_copy(accum.at[pl.ds(s * ROWS_PER_TILE, ROWS_PER_TILE)],
                    out_hbm.at[c, pl.ds(s * ROWS_PER_TILE, ROWS_PER_TILE)])


_sc_scatter = pl.kernel(
    _sc_scatter_body,
    mesh=plsc.VectorSubcoreMesh(core_axis_name="c", subcore_axis_name="s"),
    out_type=jax.ShapeDtypeStruct((2, N_PAD, F), jnp.float32),
    scratch_types=(
        [pltpu.VMEM_SHARED((N_PAD, F), jnp.float32)]
        + [pltpu.VMEM((2, EB), jnp.int32)] * 3
        + [pltpu.VMEM((EB, F), jnp.float32)] * 3
        + [pltpu.SemaphoreType.DMA] * 9
    ),
)


DF = 128


def _sc_deg_body(pair_hbm, zeros_hbm, ones_hbm, out_hbm,
                 accum, p0, p1, ones_rows, ss0, ss1, sp0, sp1):
    pairs = [p0, p1]
    sem_s = [ss0, ss1]
    sem_p = [sp0, sp1]
    c = lax.axis_index("c")
    s = lax.axis_index("s")
    wid = c * 16 + s
    base = wid * NB

    zcp = pltpu.async_copy(
        zeros_hbm.at[pl.ds(s * ROWS_PER_TILE, ROWS_PER_TILE)],
        accum.at[pl.ds(s * ROWS_PER_TILE, ROWS_PER_TILE)], ss0)
    pltpu.sync_copy(ones_hbm, ones_rows)
    pltpu.sync_copy(pair_hbm.at[base], p0)
    pltpu.async_copy(pair_hbm.at[base + 1], p1, sp1)
    zcp.wait()
    plsc.subcore_barrier()

    def scatter(k):
        pltpu.async_copy(ones_rows, accum.at[pairs[k].at[1]], sem_s[k],
                         add=True)

    def swait(k):
        pltpu.make_async_copy(ones_rows, accum.at[pairs[k].at[1]],
                              sem_s[k]).wait()

    def astage(k, batch):
        b = jnp.minimum(batch, NB - 1)
        pltpu.async_copy(pair_hbm.at[base + b], pairs[k], sem_p[k])

    def pwait(k):
        pltpu.make_async_copy(pair_hbm.at[base], pairs[k], sem_p[k]).wait()

    scatter(0)

    def body(i, carry):
        b = 2 * i
        swait(0)
        astage(0, b + 2)
        pwait(1)
        scatter(1)
        swait(1)
        astage(1, b + 3)
        pwait(0)
        scatter(0)
        return carry

    lax.fori_loop(0, NB // 2, body, 0)
    swait(0)
    pwait(1)

    plsc.subcore_barrier()
    pltpu.sync_copy(accum.at[pl.ds(s * ROWS_PER_TILE, ROWS_PER_TILE)],
                    out_hbm.at[c, pl.ds(s * ROWS_PER_TILE, ROWS_PER_TILE)])


_sc_deg = pl.kernel(
    _sc_deg_body,
    mesh=plsc.VectorSubcoreMesh(core_axis_name="c", subcore_axis_name="s"),
    out_type=jax.ShapeDtypeStruct((2, N_PAD, DF), jnp.float32),
    scratch_types=(
        [pltpu.VMEM_SHARED((N_PAD, DF), jnp.float32)]
        + [pltpu.VMEM((2, EB), jnp.int32)] * 2
        + [pltpu.VMEM((EB, DF), jnp.float32)]
        + [pltpu.SemaphoreType.DMA] * 4
    ),
)


def _mlp_body(x_ref, w1_ref, b1_ref, w2_ref, b2_ref, da_ref, db_ref,
              h0_ref, g0_ref, q_ref, dis_ref):
    deg = da_ref[:, 0:1] + db_ref[:, 0:1] + 1.0
    dis = lax.rsqrt(deg)
    h = jnp.maximum(
        jnp.dot(x_ref[...], w1_ref[...], preferred_element_type=jnp.float32)
        + b1_ref[...], 0.0)
    h = jnp.dot(h, w2_ref[...], preferred_element_type=jnp.float32) + b2_ref[...]
    h0_ref[...] = h
    g0_ref[...] = dis * h
    q_ref[...] = 1.0 / deg
    dis_ref[...] = dis


def _update_body(sa_ref, sb_ref, g_ref, base_ref, vec_ref, out_ref):
    out_ref[...] = ((1.0 - ALPHA) * vec_ref[...]
                    * (sa_ref[...] + sb_ref[...] + g_ref[...])
                    + ALPHA * base_ref[...])


_BLK = 632
_GRID = N_PAD // _BLK


def _row_spec(width):
    return pl.BlockSpec((_BLK, width), lambda i: (i, 0))


_mlp_call = pl.pallas_call(
    _mlp_body,
    grid=(_GRID,),
    in_specs=[
        _row_spec(F),
        pl.BlockSpec((F, HID), lambda i: (0, 0)),
        pl.BlockSpec((1, HID), lambda i: (0, 0)),
        pl.BlockSpec((HID, F), lambda i: (0, 0)),
        pl.BlockSpec((1, F), lambda i: (0, 0)),
        _row_spec(DF),
        _row_spec(DF),
    ],
    out_specs=[_row_spec(F), _row_spec(F), _row_spec(1), _row_spec(1)],
    out_shape=[
        jax.ShapeDtypeStruct((N_PAD, F), jnp.float32),
        jax.ShapeDtypeStruct((N_PAD, F), jnp.float32),
        jax.ShapeDtypeStruct((N_PAD, 1), jnp.float32),
        jax.ShapeDtypeStruct((N_PAD, 1), jnp.float32),
    ],
)

_update_call = pl.pallas_call(
    _update_body,
    grid=(_GRID,),
    in_specs=[_row_spec(F), _row_spec(F), _row_spec(F), _row_spec(F),
              _row_spec(1)],
    out_specs=_row_spec(F),
    out_shape=jax.ShapeDtypeStruct((N_PAD, F), jnp.float32),
)

_final_call = pl.pallas_call(
    _update_body,
    grid=(_GRID,),
    in_specs=[_row_spec(F), _row_spec(F), _row_spec(F), _row_spec(F),
              _row_spec(1)],
    out_specs=_row_spec(F),
    out_shape=jax.ShapeDtypeStruct((N_NODES, F), jnp.float32),
)


def kernel(x, edge_index, W1, b1, W2, b2):
    src = edge_index[0].astype(jnp.int32).reshape(NW, EDGES_PER_TILE)
    dst = edge_index[1].astype(jnp.int32).reshape(NW, EDGES_PER_TILE)

    n_extra = EDGES_PAD_PER_TILE - EDGES_PER_TILE
    w_ids = jnp.arange(NW, dtype=jnp.int32)[:, None]
    j_ids = jnp.arange(n_extra, dtype=jnp.int32)[None, :]
    pad_src = (w_ids * 37 + j_ids * 89) % N_NODES
    pad_dst = N_NODES + (w_ids * 13 + j_ids * 7) % (N_PAD - N_NODES)
    src_t = jnp.concatenate([src, pad_src], axis=1).reshape(NW * NB, EB)
    dst_t = jnp.concatenate([dst, pad_dst], axis=1).reshape(NW * NB, EB)
    pair = jnp.stack([src_t, dst_t], axis=1)

    zeros = jnp.zeros((N_PAD, F), jnp.float32)
    zeros16 = jnp.zeros((N_PAD, DF), jnp.float32)
    ones16 = jnp.ones((EB, DF), jnp.float32)

    deg_parts = _sc_deg(pair, zeros16, ones16)

    x_pad = jnp.zeros((N_PAD, F), x.dtype).at[:N_NODES].set(x)
    h0, g0, q, dis = _mlp_call(x_pad, W1, b1.reshape(1, HID), W2,
                               b2.reshape(1, F), deg_parts[0], deg_parts[1])

    g = g0
    for k in range(K):
        s_parts = _sc_scatter(g, pair, zeros)
        if k < K - 1:
            g = _update_call(s_parts[0], s_parts[1], g, g0, q)
        else:
            h = _final_call(s_parts[0], s_parts[1], g, h0, dis)
    return h
